# Initial kernel scaffold; baseline (speedup 1.0000x reference)
import jax
import jax.numpy as jnp
from jax.experimental import pallas as pl


def kernel(hidden_states, router_weight, gate_up_proj, gate_up_proj_bias, down_proj, down_proj_bias):
    raise NotImplementedError("write your pallas kernel here")


# speedup vs baseline: 1.8580x; 1.8580x over previous
import functools

import jax
import jax.numpy as jnp
from jax import lax
from jax.experimental import pallas as pl
from jax.experimental.pallas import tpu as pltpu

B, S, H = 1, 2048, 1024
E, I2, TOPK = 8, 2048, 2
I = I2 // 2
N = B * S
T = 256
NBLK = 24
NSLOT = NBLK * T

_NEG = -1e30


def _router_body(x_ref, rw_ref, scores_ref, tidx_ref, tval_ref):
    x = x_ref[...]
    rw = rw_ref[...]
    logits = lax.dot_general(x, rw, (((1,), (1,)), ((), ())),
                             preferred_element_type=jnp.float32)
    lane = lax.broadcasted_iota(jnp.int32, logits.shape, 1)
    lm = jnp.where(lane < E, logits, _NEG)
    m = jnp.max(lm, axis=1, keepdims=True)
    e = jnp.exp(lm - m)
    s = e / jnp.sum(e, axis=1, keepdims=True)
    v1 = jnp.max(s, axis=1, keepdims=True)
    i1 = jnp.min(jnp.where(s == v1, lane, 127), axis=1, keepdims=True)
    sm = jnp.where(lane == i1, -1.0, s)
    v2 = jnp.max(sm, axis=1, keepdims=True)
    i2 = jnp.min(jnp.where(sm == v2, lane, 127), axis=1, keepdims=True)
    scores_ref[...] = s[:, :E]
    lane8 = lax.broadcasted_iota(jnp.int32, (T, E), 1)
    tidx_ref[...] = jnp.where(lane8 == 0, i1, i2)
    tval_ref[...] = jnp.where(lane8 == 0, v1, v2)


def _router(flat, rw_pad):
    return pl.pallas_call(
        _router_body,
        grid=(N // T,),
        in_specs=[
            pl.BlockSpec((T, H), lambda i: (i, 0)),
            pl.BlockSpec((128, H), lambda i: (0, 0)),
        ],
        out_specs=[
            pl.BlockSpec((T, E), lambda i: (i, 0)),
            pl.BlockSpec((T, E), lambda i: (i, 0)),
            pl.BlockSpec((T, E), lambda i: (i, 0)),
        ],
        out_shape=[
            jax.ShapeDtypeStruct((N, E), jnp.float32),
            jax.ShapeDtypeStruct((N, E), jnp.int32),
            jax.ShapeDtypeStruct((N, E), jnp.float32),
        ],
    )(flat, rw_pad)


def _schedule(tidx, tval):
    i12 = tidx[:, :TOPK]
    v12 = tval[:, :TOPK]
    ef = i12.reshape(-1)
    oh = (ef[:, None] == jnp.arange(E)[None, :]).astype(jnp.int32)
    csum = jnp.cumsum(oh, axis=0)
    rank = jnp.sum(oh * csum, axis=1) - 1
    counts = csum[-1]
    pc = ((counts + T - 1) // T) * T
    ends = jnp.cumsum(pc)
    off = ends - pc
    slot = off[ef] + rank
    nblocks = (ends[-1] // T).astype(jnp.int32)
    tok = jnp.arange(TOPK * N, dtype=jnp.int32) // TOPK
    sorted_token = jnp.zeros((NSLOT,), jnp.int32).at[slot].set(tok)
    slot_w = jnp.zeros((NSLOT,), jnp.float32).at[slot].set(v12.reshape(-1))
    bstart = jnp.arange(NBLK, dtype=jnp.int32) * T
    raw = jnp.clip(jnp.searchsorted(ends, bstart, side='right'), 0, E - 1)
    last = raw[nblocks - 1]
    block_expert = jnp.where(jnp.arange(NBLK) < nblocks, raw, last)
    block_expert = block_expert.astype(jnp.int32)
    s1 = slot.reshape(N, TOPK)[:, 0]
    s2 = slot.reshape(N, TOPK)[:, 1]
    return (sorted_token, slot_w.reshape(NSLOT, 1), block_expert,
            jnp.reshape(nblocks, (1,)), s1, s2)


def _mlp_body(nb_ref, be_ref, xs_ref, w1_ref, b1_ref, w2_ref, b2_ref,
              sw_ref, y_ref):
    b = pl.program_id(0)

    @pl.when(b < nb_ref[0])
    def _():
        x = xs_ref[...]
        gu = lax.dot_general(x, w1_ref[0], (((1,), (0,)), ((), ())),
                             preferred_element_type=jnp.float32)
        gu = gu + b1_ref[0]
        gate = jnp.minimum(gu[:, :I], 7.0)
        up = jnp.clip(gu[:, I:], -7.0, 7.0)
        act = (up + 1.0) * gate * (1.0 / (1.0 + jnp.exp(gate * -1.702)))
        y = lax.dot_general(act, w2_ref[0], (((1,), (0,)), ((), ())),
                            preferred_element_type=jnp.float32)
        y = y + b2_ref[0]
        y_ref[...] = y * sw_ref[...]


def _grouped_mlp(xs, w1p, b1p, w2, b2, slot_w, block_expert, nblocks):
    grid_spec = pltpu.PrefetchScalarGridSpec(
        num_scalar_prefetch=2,
        grid=(NBLK,),
        in_specs=[
            pl.BlockSpec((T, H), lambda b, nb, be: (b, 0)),
            pl.BlockSpec((1, H, I2), lambda b, nb, be: (be[b], 0, 0)),
            pl.BlockSpec((1, 1, I2), lambda b, nb, be: (be[b], 0, 0)),
            pl.BlockSpec((1, I, H), lambda b, nb, be: (be[b], 0, 0)),
            pl.BlockSpec((1, 1, H), lambda b, nb, be: (be[b], 0, 0)),
            pl.BlockSpec((T, 1), lambda b, nb, be: (b, 0)),
        ],
        out_specs=pl.BlockSpec((T, H), lambda b, nb, be: (b, 0)),
    )
    return pl.pallas_call(
        _mlp_body,
        grid_spec=grid_spec,
        out_shape=jax.ShapeDtypeStruct((NSLOT, H), jnp.float32),
    )(nblocks, block_expert, xs, w1p, b1p, w2, b2, slot_w)


def kernel(hidden_states, router_weight, gate_up_proj, gate_up_proj_bias,
           down_proj, down_proj_bias):
    flat = hidden_states.reshape(N, H)
    rw_pad = jnp.zeros((128, H), jnp.float32).at[:E].set(router_weight)
    w1p = jnp.concatenate([gate_up_proj[:, :, ::2], gate_up_proj[:, :, 1::2]],
                          axis=2)
    b1p = jnp.concatenate([gate_up_proj_bias[:, ::2], gate_up_proj_bias[:, 1::2]],
                          axis=1).reshape(E, 1, I2)
    b2 = down_proj_bias.reshape(E, 1, H)

    scores, tidx, tval = _router(flat, rw_pad)
    sorted_token, slot_w, block_expert, nblocks, s1, s2 = _schedule(tidx, tval)

    xs = flat[sorted_token]
    y = _grouped_mlp(xs, w1p, b1p, down_proj, b2, slot_w, block_expert,
                     nblocks)
    out = y[s1] + y[s2]
    return out.reshape(B, S, H), scores


# speedup vs baseline: 3.2360x; 1.7417x over previous
import functools

import jax
import jax.numpy as jnp
from jax import lax
from jax.experimental import pallas as pl
from jax.experimental.pallas import tpu as pltpu

B, S, H = 1, 2048, 1024
E, I2, TOPK = 8, 2048, 2
I = I2 // 2
N = B * S
T = 256
NBLK = 24
NSLOT = NBLK * T

_NEG = -1e30


def _router_body(x_ref, rw_ref, scores_ref, tidx_ref, tval_ref):
    x = x_ref[...]
    rw = rw_ref[...]
    logits = lax.dot_general(x, rw, (((1,), (1,)), ((), ())),
                             preferred_element_type=jnp.float32)
    lane = lax.broadcasted_iota(jnp.int32, logits.shape, 1)
    lm = jnp.where(lane < E, logits, _NEG)
    m = jnp.max(lm, axis=1, keepdims=True)
    e = jnp.exp(lm - m)
    s = e / jnp.sum(e, axis=1, keepdims=True)
    v1 = jnp.max(s, axis=1, keepdims=True)
    i1 = jnp.min(jnp.where(s == v1, lane, 127), axis=1, keepdims=True)
    sm = jnp.where(lane == i1, -1.0, s)
    v2 = jnp.max(sm, axis=1, keepdims=True)
    i2 = jnp.min(jnp.where(sm == v2, lane, 127), axis=1, keepdims=True)
    scores_ref[...] = s[:, :E]
    lane8 = lax.broadcasted_iota(jnp.int32, (T, E), 1)
    tidx_ref[...] = jnp.where(lane8 == 0, i1, i2)
    tval_ref[...] = jnp.where(lane8 == 0, v1, v2)


def _router(flat, rw_pad):
    return pl.pallas_call(
        _router_body,
        grid=(N // T,),
        in_specs=[
            pl.BlockSpec((T, H), lambda i: (i, 0)),
            pl.BlockSpec((128, H), lambda i: (0, 0)),
        ],
        out_specs=[
            pl.BlockSpec((T, E), lambda i: (i, 0)),
            pl.BlockSpec((T, E), lambda i: (i, 0)),
            pl.BlockSpec((T, E), lambda i: (i, 0)),
        ],
        out_shape=[
            jax.ShapeDtypeStruct((N, E), jnp.float32),
            jax.ShapeDtypeStruct((N, E), jnp.int32),
            jax.ShapeDtypeStruct((N, E), jnp.float32),
        ],
    )(flat, rw_pad)


def _schedule(tidx, tval):
    i12 = tidx[:, :TOPK]
    v12 = tval[:, :TOPK]
    ef = i12.reshape(-1)
    oh = (ef[:, None] == jnp.arange(E)[None, :]).astype(jnp.int32)
    csum = jnp.cumsum(oh, axis=0)
    rank = jnp.sum(oh * csum, axis=1) - 1
    counts = csum[-1]
    pc = ((counts + T - 1) // T) * T
    ends = jnp.cumsum(pc)
    off = ends - pc
    slot = off[ef] + rank
    nblocks = (ends[-1] // T).astype(jnp.int32)
    tok = jnp.arange(TOPK * N, dtype=jnp.int32) // TOPK
    sorted_token = jnp.zeros((NSLOT,), jnp.int32).at[slot].set(tok)
    slot_w = jnp.zeros((NSLOT,), jnp.float32).at[slot].set(v12.reshape(-1))
    bstart = jnp.arange(NBLK, dtype=jnp.int32) * T
    raw = jnp.clip(jnp.searchsorted(ends, bstart, side='right'), 0, E - 1)
    last = raw[nblocks - 1]
    block_expert = jnp.where(jnp.arange(NBLK) < nblocks, raw, last)
    block_expert = block_expert.astype(jnp.int32)
    s1 = slot.reshape(N, TOPK)[:, 0]
    s2 = slot.reshape(N, TOPK)[:, 1]
    return (sorted_token, slot_w.reshape(NSLOT, 1), block_expert,
            jnp.reshape(nblocks, (1,)), s1, s2)


def _mlp_body(nb_ref, be_ref, xs_ref, w1_ref, b1_ref, w2_ref, b2_ref,
              sw_ref, y_ref):
    b = pl.program_id(0)

    @pl.when(b < nb_ref[0])
    def _():
        x = xs_ref[...].astype(jnp.bfloat16)
        gu = lax.dot_general(x, w1_ref[0], (((1,), (0,)), ((), ())),
                             preferred_element_type=jnp.float32)
        gu = gu + b1_ref[0]
        gate = jnp.minimum(gu[:, :I], 7.0)
        up = jnp.clip(gu[:, I:], -7.0, 7.0)
        act = (up + 1.0) * gate * (1.0 / (1.0 + jnp.exp(gate * -1.702)))
        y = lax.dot_general(act.astype(jnp.bfloat16), w2_ref[0],
                            (((1,), (0,)), ((), ())),
                            preferred_element_type=jnp.float32)
        y = y + b2_ref[0]
        y_ref[...] = y * sw_ref[...]


def _grouped_mlp(xs, w1p, b1p, w2, b2, slot_w, block_expert, nblocks):
    grid_spec = pltpu.PrefetchScalarGridSpec(
        num_scalar_prefetch=2,
        grid=(NBLK,),
        in_specs=[
            pl.BlockSpec((T, H), lambda b, nb, be: (b, 0)),
            pl.BlockSpec((1, H, I2), lambda b, nb, be: (be[b], 0, 0)),

            pl.BlockSpec((1, 1, I2), lambda b, nb, be: (be[b], 0, 0)),
            pl.BlockSpec((1, I, H), lambda b, nb, be: (be[b], 0, 0)),
            pl.BlockSpec((1, 1, H), lambda b, nb, be: (be[b], 0, 0)),
            pl.BlockSpec((T, 1), lambda b, nb, be: (b, 0)),
        ],
        out_specs=pl.BlockSpec((T, H), lambda b, nb, be: (b, 0)),
    )
    return pl.pallas_call(
        _mlp_body,
        grid_spec=grid_spec,
        out_shape=jax.ShapeDtypeStruct((NSLOT, H), jnp.float32),
    )(nblocks, block_expert, xs, w1p, b1p, w2, b2, slot_w)


def kernel(hidden_states, router_weight, gate_up_proj, gate_up_proj_bias,
           down_proj, down_proj_bias):
    flat = hidden_states.reshape(N, H)
    rw_pad = jnp.zeros((128, H), jnp.float32).at[:E].set(router_weight)
    w1p = jnp.concatenate([gate_up_proj[:, :, ::2], gate_up_proj[:, :, 1::2]],
                          axis=2).astype(jnp.bfloat16)
    w2b = down_proj.astype(jnp.bfloat16)
    b1p = jnp.concatenate([gate_up_proj_bias[:, ::2], gate_up_proj_bias[:, 1::2]],
                          axis=1).reshape(E, 1, I2)
    b2 = down_proj_bias.reshape(E, 1, H)

    scores, tidx, tval = _router(flat, rw_pad)
    sorted_token, slot_w, block_expert, nblocks, s1, s2 = _schedule(tidx, tval)

    xs = flat[sorted_token]
    y = _grouped_mlp(xs, w1p, b1p, w2b, b2, slot_w, block_expert,
                     nblocks)
    out = y[s1] + y[s2]
    return out.reshape(B, S, H), scores
